# Initial kernel scaffold; baseline (speedup 1.0000x reference)
import jax
import jax.numpy as jnp
from jax.experimental import pallas as pl


def kernel(x, position_embedding):
    raise NotImplementedError("write your pallas kernel here")


# speedup vs baseline: 1.6532x; 1.6532x over previous
import functools

import jax
import jax.numpy as jnp
from jax import lax
from jax.experimental import pallas as pl
from jax.experimental.pallas import tpu as pltpu
from jax.experimental.pallas import tpu_sc as plsc


def _make_sc_broadcast(B: int, S: int, D: int, dtype):
    info = plsc.get_sparse_core_info()
    NC, NS = info.num_cores, info.num_subcores
    NW = NC * NS
    assert S % NW == 0
    rows_per_w = S // NW
    chunk = min(32, rows_per_w)
    assert rows_per_w % chunk == 0
    n_chunks = rows_per_w // chunk

    mesh = plsc.VectorSubcoreMesh(core_axis_name="c", subcore_axis_name="s")

    @functools.partial(
        pl.kernel,
        mesh=mesh,
        out_type=jax.ShapeDtypeStruct((B, S, D), dtype),
        scratch_types=[pltpu.VMEM((chunk, D), dtype)],
    )
    def broadcast_rows(table_hbm, out_hbm, buf):
        wid = lax.axis_index("s") * NC + lax.axis_index("c")
        base = wid * rows_per_w
        for j in range(n_chunks):
            r0 = base + j * chunk
            pltpu.sync_copy(table_hbm.at[pl.ds(r0, chunk), :], buf)
            for b in range(B):
                pltpu.sync_copy(buf, out_hbm.at[b, pl.ds(r0, chunk), :])

    return broadcast_rows


def kernel(x, position_embedding):
    B, S, _ = x.shape
    _, D = position_embedding.shape
    fn = _make_sc_broadcast(B, S, D, position_embedding.dtype)
    return fn(position_embedding)
